# baseline (speedup 1.0000x reference)
import functools

import jax
import jax.numpy as jnp
from jax import lax
from jax.experimental import pallas as pl
from jax.experimental.pallas import tpu as pltpu
from jax.experimental.pallas import tpu_sc as plsc

_NUM_SAMPLES = 10000
_K = 80


def _sc_partial_segment_sum(emb, ids, zeros, *, n, d, sp):
    nc, ns = 2, 16
    nw = nc * ns
    cn = n // nw
    nblk = cn // _K
    gs = sp // ns
    nbuf = 4
    mesh = plsc.VectorSubcoreMesh(core_axis_name="c", subcore_axis_name="s")

    @functools.partial(
        pl.kernel,
        out_type=jax.ShapeDtypeStruct((nc, sp, d), jnp.float32),
        mesh=mesh,
        scratch_types=[
            [pltpu.VMEM((_K, d), jnp.float32) for _ in range(nbuf)],
            [pltpu.VMEM((_K,), jnp.int32) for _ in range(nbuf)],
            pltpu.VMEM_SHARED((sp, d), jnp.float32),
            [pltpu.SemaphoreType.DMA for _ in range(nbuf)],
            [pltpu.SemaphoreType.DMA for _ in range(nbuf)],
            [pltpu.SemaphoreType.DMA for _ in range(nbuf)],
        ],
    )
    def k(emb_hbm, ids_hbm, zeros_hbm, out_hbm,
          rows, idxs, acc, lsem, isem, ssem):
        c = lax.axis_index("c")
        sub = lax.axis_index("s")
        wid = c * ns + sub

        def row_desc(j, blk):
            src = emb_hbm.at[pl.ds(wid * cn + blk * _K, _K)]
            return pltpu.make_async_copy(src, rows[j], lsem[j])

        def idx_desc(j, blk):
            src = ids_hbm.at[pl.ds(wid * cn + blk * _K, _K)]
            return pltpu.make_async_copy(src, idxs[j], isem[j])

        def load_start(j, blk):
            row_desc(j, blk).start()
            idx_desc(j, blk).start()

        def load_wait(j, blk):
            row_desc(j, blk).wait()
            idx_desc(j, blk).wait()

        def scat_start(j):
            pltpu.async_copy(rows[j], acc.at[idxs[j]], ssem[j], add=True)

        def scat_wait(j):
            pltpu.make_async_copy(rows[j], acc.at[idxs[j]], ssem[j]).wait()

        load_start(0, 0)
        pltpu.sync_copy(zeros_hbm, acc.at[pl.ds(sub * gs, gs)])
        plsc.subcore_barrier()

        def group(g, carry):
            for j in range(nbuf):
                i = nbuf * g + j
                load_wait(j, i)
                scat_start(j)
                jj = (j + 1) % nbuf

                @pl.when(i >= 3)
                def _():
                    scat_wait(jj)

                @pl.when(i + 1 <= nblk - 1)
                def _():
                    load_start(jj, i + 1)
            return carry

        lax.fori_loop(0, (nblk - 1) // nbuf, group, 0)
        last = nblk - 1
        load_wait(last % nbuf, last)
        scat_start(last % nbuf)
        scat_wait((last - 3) % nbuf)
        scat_wait((last - 2) % nbuf)
        scat_wait((last - 1) % nbuf)
        scat_wait(last % nbuf)

        plsc.subcore_barrier()
        pltpu.sync_copy(acc.at[pl.ds(sub * gs, gs)],
                        out_hbm.at[c, pl.ds(sub * gs, gs)])

    return k(emb, ids, zeros)


def _merge_body(p_ref, o_ref):
    o_ref[...] = p_ref[0] + p_ref[1]


def kernel(element_embeddings, element_to_sample_map, num_samples):
    n, d = element_embeddings.shape
    s = _NUM_SAMPLES
    sp = 10240
    ids = element_to_sample_map.astype(jnp.int32)
    zeros = jnp.zeros((sp // 16, d), jnp.float32)
    partials = _sc_partial_segment_sum(element_embeddings, ids, zeros,
                                       n=n, d=d, sp=sp)
    blk = s // 10
    return pl.pallas_call(
        _merge_body,
        out_shape=jax.ShapeDtypeStruct((s, d), jnp.float32),
        grid=(10,),
        in_specs=[pl.BlockSpec((2, blk, d), lambda i: (0, i, 0))],
        out_specs=pl.BlockSpec((blk, d), lambda i: (i, 0)),
    )(partials)


# speedup vs baseline: 1.2695x; 1.2695x over previous
import functools

import jax
import jax.numpy as jnp
from jax import lax
from jax.experimental import pallas as pl
from jax.experimental.pallas import tpu as pltpu
from jax.experimental.pallas import tpu_sc as plsc

_NUM_SAMPLES = 10000
_K = 80


def _sc_partial_segment_sum(emb, ids, zeros, *, n, d, sp):
    nc, ns = 2, 16
    nw = nc * ns
    cn = n // nw
    nblk = cn // _K
    gs = sp // ns
    nbuf = 4
    mesh = plsc.VectorSubcoreMesh(core_axis_name="c", subcore_axis_name="s")

    @functools.partial(
        pl.kernel,
        out_type=jax.ShapeDtypeStruct((nc, sp, d), jnp.float32),
        mesh=mesh,
        scratch_types=[
            [pltpu.VMEM((_K, d), jnp.float32) for _ in range(nbuf)],
            [pltpu.VMEM((_K,), jnp.int32) for _ in range(nbuf)],
            pltpu.VMEM_SHARED((sp, d), jnp.float32),
            [pltpu.SemaphoreType.DMA for _ in range(nbuf)],
            [pltpu.SemaphoreType.DMA for _ in range(nbuf)],
            [pltpu.SemaphoreType.DMA for _ in range(nbuf)],
        ],
    )
    def k(emb_hbm, ids_hbm, zeros_hbm, out_hbm,
          rows, idxs, acc, lsem, isem, ssem):
        c = lax.axis_index("c")
        sub = lax.axis_index("s")
        wid = c * ns + sub

        def row_desc(j, blk):
            src = emb_hbm.at[pl.ds(wid * cn + blk * _K, _K)]
            return pltpu.make_async_copy(src, rows[j], lsem[j])

        def idx_desc(j, blk):
            src = ids_hbm.at[pl.ds(wid * cn + blk * _K, _K)]
            return pltpu.make_async_copy(src, idxs[j], isem[j])

        def load_start(j, blk):
            row_desc(j, blk).start()
            idx_desc(j, blk).start()

        def load_wait(j, blk):
            row_desc(j, blk).wait()
            idx_desc(j, blk).wait()

        def scat_start(j):
            pltpu.async_copy(rows[j], acc.at[idxs[j]], ssem[j], add=True)

        def scat_wait(j):
            pltpu.make_async_copy(rows[j], acc.at[idxs[j]], ssem[j]).wait()

        load_start(0, 0)
        load_start(1, 1)
        pltpu.sync_copy(zeros_hbm, acc.at[pl.ds(sub * gs, gs)])
        plsc.subcore_barrier()

        def group(g, carry):
            for j in range(nbuf):
                i = nbuf * g + j
                load_wait(j, i)
                scat_start(j)
                jj = (j + 2) % nbuf

                @pl.when(i >= 2)
                def _():
                    scat_wait(jj)

                @pl.when(i + 2 <= nblk - 1)
                def _():
                    load_start(jj, i + 2)
            return carry

        lax.fori_loop(0, (nblk - 1) // nbuf, group, 0)
        last = nblk - 1
        load_wait(last % nbuf, last)
        scat_start(last % nbuf)
        scat_wait((last - 2) % nbuf)
        scat_wait((last - 1) % nbuf)
        scat_wait(last % nbuf)

        plsc.subcore_barrier()
        pltpu.sync_copy(acc.at[pl.ds(sub * gs, gs)],
                        out_hbm.at[c, pl.ds(sub * gs, gs)])

    return k(emb, ids, zeros)


def _merge_body(p_ref, o_ref):
    o_ref[...] = p_ref[0] + p_ref[1]


def kernel(element_embeddings, element_to_sample_map, num_samples):
    n, d = element_embeddings.shape
    s = _NUM_SAMPLES
    sp = 10240
    ids = element_to_sample_map.astype(jnp.int32)
    zeros = jnp.zeros((sp // 16, d), jnp.float32)
    partials = _sc_partial_segment_sum(element_embeddings, ids, zeros,
                                       n=n, d=d, sp=sp)
    blk = s // 10
    return pl.pallas_call(
        _merge_body,
        out_shape=jax.ShapeDtypeStruct((s, d), jnp.float32),
        grid=(10,),
        in_specs=[pl.BlockSpec((2, blk, d), lambda i: (0, i, 0))],
        out_specs=pl.BlockSpec((blk, d), lambda i: (i, 0)),
    )(partials)


# speedup vs baseline: 1.2716x; 1.0016x over previous
import functools

import jax
import jax.numpy as jnp
from jax import lax
from jax.experimental import pallas as pl
from jax.experimental.pallas import tpu as pltpu
from jax.experimental.pallas import tpu_sc as plsc

_NUM_SAMPLES = 10000
_K = 128
_HALF = 5120
_ACC = 5248
_SLOTS = 160


def _sc_range_segment_sum(emb, ids, splitv, zeros, *, n, d):
    nc, ns = 2, 16
    gs = _ACC // ns
    ws = _HALF // ns
    nbuf = 4
    mesh = plsc.VectorSubcoreMesh(core_axis_name="c", subcore_axis_name="s")

    @functools.partial(
        pl.kernel,
        out_type=jax.ShapeDtypeStruct((2 * _HALF, d), jnp.float32),
        mesh=mesh,
        scratch_types=[
            [pltpu.VMEM((_K, d), jnp.float32) for _ in range(nbuf)],
            [pltpu.VMEM((_K,), jnp.int32) for _ in range(nbuf)],
            pltpu.VMEM((16,), jnp.int32),
            pltpu.VMEM_SHARED((_ACC, d), jnp.float32),
            [pltpu.SemaphoreType.DMA for _ in range(nbuf)],
            [pltpu.SemaphoreType.DMA for _ in range(nbuf)],
            [pltpu.SemaphoreType.DMA for _ in range(nbuf)],
        ],
    )
    def k(emb_hbm, ids_hbm, splitv_hbm, zeros_hbm, out_hbm,
          rows, idxs, splv, acc, lsem, isem, ssem):
        c = lax.axis_index("c")
        sub = lax.axis_index("s")

        pltpu.sync_copy(splitv_hbm, splv)
        split = splv[...][0]

        lo = jnp.where(c == 0, 0, split)
        hi = jnp.where(c == 0, split, n)
        nb_t = (hi - lo + ns * _K - 1) // (ns * _K)
        a_t = lo + sub * nb_t * _K
        b_t = jnp.minimum(a_t + nb_t * _K, hi)
        start = a_t - lax.rem(a_t, 8)
        seg_base = c * _HALF

        def blk_base(blk):
            return pl.multiple_of(jnp.minimum(start + blk * _K, n - _K), 8)

        def row_desc(j, blk):
            src = emb_hbm.at[pl.ds(blk_base(blk), _K)]
            return pltpu.make_async_copy(src, rows[j], lsem[j])

        def idx_desc(j, blk):
            src = ids_hbm.at[pl.ds(blk_base(blk), _K)]
            return pltpu.make_async_copy(src, idxs[j], isem[j])

        def load_start(j, blk):
            row_desc(j, blk).start()
            idx_desc(j, blk).start()

        def load_wait(j, blk):
            row_desc(j, blk).wait()
            idx_desc(j, blk).wait()

        def mask_idx(j, blk):
            base = blk_base(blk)
            u = start + blk * _K
            for g in range(_K // 16):
                v = idxs[j][pl.ds(g * 16, 16)]
                gid = jax.lax.broadcasted_iota(jnp.int32, (16,), 0) + (
                    base + g * 16)
                valid = (gid >= a_t) & (gid < b_t) & (gid >= u)
                lid = jnp.where(valid, v - seg_base, _HALF)
                idxs[j][pl.ds(g * 16, 16)] = lid

        def scat_start(j):
            pltpu.async_copy(rows[j], acc.at[idxs[j]], ssem[j], add=True)

        def scat_wait(j):
            pltpu.make_async_copy(rows[j], acc.at[idxs[j]], ssem[j]).wait()

        load_start(0, 0)

        @pl.when(nb_t >= 1)
        def _():
            load_start(1, 1)

        pltpu.sync_copy(zeros_hbm, acc.at[pl.ds(sub * gs, gs)])
        plsc.subcore_barrier()

        def group(g, carry):
            for j in range(nbuf):
                i = nbuf * g + j
                jj = (j + 2) % nbuf

                @pl.when(i <= nb_t)
                def _():
                    load_wait(j, i)
                    mask_idx(j, i)
                    scat_start(j)

                @pl.when((i >= 2) & (i - 2 <= nb_t))
                def _():
                    scat_wait(jj)

                @pl.when((i + 2 >= 2) & (i + 2 <= nb_t))
                def _():
                    load_start(jj, i + 2)
            return carry

        lax.fori_loop(0, _SLOTS // nbuf, group, 0)

        plsc.subcore_barrier()
        pltpu.sync_copy(acc.at[pl.ds(sub * ws, ws)],
                        out_hbm.at[pl.ds(c * _HALF + sub * ws, ws)])

    return k(emb, ids, splitv, zeros)


def _crop_body(x_ref, o_ref):
    o_ref[...] = x_ref[...]


def kernel(element_embeddings, element_to_sample_map, num_samples):
    n, d = element_embeddings.shape
    s = _NUM_SAMPLES
    ids = element_to_sample_map.astype(jnp.int32)
    split = jnp.sum((ids < _HALF).astype(jnp.int32))
    splitv = jnp.full((16,), split, jnp.int32)
    zeros = jnp.zeros((_ACC // 16, d), jnp.float32)
    padded = _sc_range_segment_sum(element_embeddings, ids, splitv, zeros,
                                   n=n, d=d)
    blk = s // 10
    return pl.pallas_call(
        _crop_body,
        out_shape=jax.ShapeDtypeStruct((s, d), jnp.float32),
        grid=(10,),
        in_specs=[pl.BlockSpec((blk, d), lambda i: (i, 0))],
        out_specs=pl.BlockSpec((blk, d), lambda i: (i, 0)),
    )(padded)


# speedup vs baseline: 1.3561x; 1.0665x over previous
import functools

import jax
import jax.numpy as jnp
from jax import lax
from jax.experimental import pallas as pl
from jax.experimental.pallas import tpu as pltpu
from jax.experimental.pallas import tpu_sc as plsc

_NUM_SAMPLES = 10000
_K = 128
_HALF = 5120
_ACC = 5248
_SLOTS = 160


def _sc_range_segment_sum(emb, ids, splitv, zeros, *, n, d, s):
    nc, ns = 2, 16
    gs = _ACC // ns
    ws = _HALF // ns
    w1 = (s - _HALF) // ns - ((s - _HALF) // ns) % 8
    nbuf = 4
    mesh = plsc.VectorSubcoreMesh(core_axis_name="c", subcore_axis_name="s")

    @functools.partial(
        pl.kernel,
        out_type=jax.ShapeDtypeStruct((s, d), jnp.float32),
        mesh=mesh,
        scratch_types=[
            [pltpu.VMEM((_K, d), jnp.float32) for _ in range(nbuf)],
            [pltpu.VMEM((_K,), jnp.int32) for _ in range(nbuf)],
            pltpu.VMEM((16,), jnp.int32),
            pltpu.VMEM_SHARED((_ACC, d), jnp.float32),
            [pltpu.SemaphoreType.DMA for _ in range(nbuf)],
            [pltpu.SemaphoreType.DMA for _ in range(nbuf)],
            [pltpu.SemaphoreType.DMA for _ in range(nbuf)],
        ],
    )
    def k(emb_hbm, ids_hbm, splitv_hbm, zeros_hbm, out_hbm,
          rows, idxs, splv, acc, lsem, isem, ssem):
        c = lax.axis_index("c")
        sub = lax.axis_index("s")

        pltpu.sync_copy(splitv_hbm, splv)
        split = splv[...][0]

        lo = jnp.where(c == 0, 0, split)
        hi = jnp.where(c == 0, split, n)
        nb_t = (hi - lo + ns * _K - 1) // (ns * _K)
        a_t = lo + sub * nb_t * _K
        b_t = jnp.minimum(a_t + nb_t * _K, hi)
        start = a_t - lax.rem(a_t, 8)
        seg_base = c * _HALF

        def blk_base(blk):
            return pl.multiple_of(jnp.minimum(start + blk * _K, n - _K), 8)

        def row_desc(j, blk):
            src = emb_hbm.at[pl.ds(blk_base(blk), _K)]
            return pltpu.make_async_copy(src, rows[j], lsem[j])

        def idx_desc(j, blk):
            src = ids_hbm.at[pl.ds(blk_base(blk), _K)]
            return pltpu.make_async_copy(src, idxs[j], isem[j])

        def load_start(j, blk):
            row_desc(j, blk).start()
            idx_desc(j, blk).start()

        def load_wait(j, blk):
            row_desc(j, blk).wait()
            idx_desc(j, blk).wait()

        def mask_idx(j, blk):
            base = blk_base(blk)
            u = start + blk * _K
            for g in range(_K // 16):
                v = idxs[j][pl.ds(g * 16, 16)]
                gid = jax.lax.broadcasted_iota(jnp.int32, (16,), 0) + (
                    base + g * 16)
                valid = (gid >= a_t) & (gid < b_t) & (gid >= u)
                lid = jnp.where(valid, v - seg_base, _HALF)
                idxs[j][pl.ds(g * 16, 16)] = lid

        def scat_start(j):
            pltpu.async_copy(rows[j], acc.at[idxs[j]], ssem[j], add=True)

        def scat_wait(j):
            pltpu.make_async_copy(rows[j], acc.at[idxs[j]], ssem[j]).wait()

        load_start(0, 0)

        @pl.when(nb_t >= 1)
        def _():
            load_start(1, 1)

        pltpu.sync_copy(zeros_hbm, acc.at[pl.ds(sub * gs, gs)])
        plsc.subcore_barrier()

        def group(g, carry):
            for j in range(nbuf):
                i = nbuf * g + j
                jj = (j + 2) % nbuf

                @pl.when(i <= nb_t)
                def _():
                    load_wait(j, i)
                    mask_idx(j, i)
                    scat_start(j)

                @pl.when((i >= 2) & (i - 2 <= nb_t))
                def _():
                    scat_wait(jj)

                @pl.when((i + 2 >= 2) & (i + 2 <= nb_t))
                def _():
                    load_start(jj, i + 2)
            return carry

        lax.fori_loop(0, _SLOTS // nbuf, group, 0)

        plsc.subcore_barrier()
        tail = (s - _HALF) - (ns - 1) * w1

        @pl.when(c == 0)
        def _():
            pltpu.sync_copy(acc.at[pl.ds(sub * ws, ws)],
                            out_hbm.at[pl.ds(sub * ws, ws)])

        @pl.when((c == 1) & (sub < ns - 1))
        def _():
            pltpu.sync_copy(acc.at[pl.ds(sub * w1, w1)],
                            out_hbm.at[pl.ds(_HALF + sub * w1, w1)])

        @pl.when((c == 1) & (sub == ns - 1))
        def _():
            pltpu.sync_copy(acc.at[pl.ds((ns - 1) * w1, tail)],
                            out_hbm.at[pl.ds(_HALF + (ns - 1) * w1, tail)])

    return k(emb, ids, splitv, zeros)


def kernel(element_embeddings, element_to_sample_map, num_samples):
    n, d = element_embeddings.shape
    s = _NUM_SAMPLES
    ids = element_to_sample_map.astype(jnp.int32)
    split = jnp.sum((ids < _HALF).astype(jnp.int32))
    splitv = jnp.full((16,), split, jnp.int32)
    zeros = jnp.zeros((_ACC // 16, d), jnp.float32)
    return _sc_range_segment_sum(element_embeddings, ids, splitv, zeros,
                                 n=n, d=d, s=s)
